# baseline (speedup 1.0000x reference)
import functools

import jax
import jax.numpy as jnp
from jax import lax
from jax.experimental import pallas as pl
from jax.experimental.pallas import tpu as pltpu
from jax.experimental.pallas import tpu_sc as plsc

_IDX_CHUNK = 128


@functools.lru_cache(maxsize=None)
def _build_embed_kernel(B, V, D):
    info = plsc.get_sparse_core_info()
    nw = info.num_cores * info.num_subcores
    b_per_w = B // nw
    chunks = b_per_w // _IDX_CHUNK
    assert b_per_w % _IDX_CHUNK == 0

    mesh = plsc.VectorSubcoreMesh(core_axis_name="c", subcore_axis_name="s")

    @functools.partial(
        pl.kernel,
        mesh=mesh,
        compiler_params=pltpu.CompilerParams(use_tc_tiling_on_sc=False),
        out_type=jax.ShapeDtypeStruct((B, D), jnp.float32),
        scratch_types=[
            pltpu.VMEM((chunks, _IDX_CHUNK), jnp.int32),
            pltpu.VMEM((chunks, _IDX_CHUNK, D), jnp.float32),
            pltpu.SemaphoreType.DMA,
        ],
    )
    def embed(idx_hbm, table_hbm, out_hbm, idx_v, rows_v, sem):
        wid = lax.axis_index("s") * info.num_cores + lax.axis_index("c")
        base = wid * b_per_w
        pltpu.sync_copy(idx_hbm.at[wid], idx_v)
        copies = [
            pltpu.async_copy(table_hbm.at[idx_v.at[j]], rows_v.at[j], sem)
            for j in range(chunks)
        ]
        for j in range(chunks):
            copies[j].wait()
            pltpu.sync_copy(
                rows_v.at[j], out_hbm.at[pl.ds(base + j * _IDX_CHUNK, _IDX_CHUNK)]
            )

    return embed


def kernel(class_labels, table):
    B = class_labels.shape[0]
    V, D = table.shape
    info = plsc.get_sparse_core_info()
    nw = info.num_cores * info.num_subcores
    embed = _build_embed_kernel(B, V, D)
    idx = class_labels.astype(jnp.int32).reshape(nw, (B // nw) // _IDX_CHUNK, _IDX_CHUNK)
    out = embed(idx, table)
    return out[:, None, :]


# speedup vs baseline: 2.5777x; 2.5777x over previous
import functools

import jax
import jax.numpy as jnp
from jax import lax
from jax.experimental import pallas as pl
from jax.experimental.pallas import tpu as pltpu
from jax.experimental.pallas import tpu_sc as plsc


@functools.lru_cache(maxsize=None)
def _build_embed_kernel(B, V, D):
    info = plsc.get_sparse_core_info()
    nw = info.num_cores * info.num_subcores
    b_per_w = B // nw

    mesh = plsc.VectorSubcoreMesh(core_axis_name="c", subcore_axis_name="s")

    @functools.partial(
        pl.kernel,
        mesh=mesh,
        compiler_params=pltpu.CompilerParams(needs_layout_passes=False),
        out_type=jax.ShapeDtypeStruct((B, D), jnp.float32),
        scratch_types=[
            pltpu.VMEM((b_per_w,), jnp.int32),
            pltpu.VMEM((b_per_w, D), jnp.float32),
            pltpu.SemaphoreType.DMA,
        ],
    )
    def embed(idx_hbm, table_hbm, out_hbm, lab_v, rows_v, sem):
        wid = lax.axis_index("s") * info.num_cores + lax.axis_index("c")
        base = wid * b_per_w
        pltpu.sync_copy(idx_hbm.at[pl.ds(base, b_per_w)], lab_v)

        def group_body(g, carry):
            off = g * 16
            labs = lab_v[pl.ds(off, 16)]
            t_vec = lax.shift_right_logical(labs, 3)
            s_vec = lax.bitwise_and(labs, 7)
            for k in range(16):
                t = t_vec[k]
                s = s_vec[k]
                pltpu.async_copy(table_hbm.at[t, s], rows_v.at[off + k], sem)
            return carry

        lax.fori_loop(0, b_per_w // 16, group_body, 0)
        pltpu.make_async_copy(
            out_hbm.at[pl.ds(base, b_per_w)], rows_v, sem
        ).wait()
        pltpu.sync_copy(rows_v, out_hbm.at[pl.ds(base, b_per_w)])

    return embed


def kernel(class_labels, table):
    B = class_labels.shape[0]
    V, D = table.shape
    embed = _build_embed_kernel(B, V, D)
    t3 = table.reshape(V // 8, 8, D)
    out = embed(class_labels.astype(jnp.int32), t3)
    return out[:, None, :]
